# Initial kernel scaffold; baseline (speedup 1.0000x reference)
import jax
import jax.numpy as jnp
from jax.experimental import pallas as pl


def kernel(feat_company, feat_brand, feat_organize, edge_bc, edge_oc, W_emb_c, W_emb_b, W_emb_o, W_self_1, W_b_1, W_o_1, W_self_2, W_b_2, W_o_2, W1, b1, W2, b2, W3, b3):
    raise NotImplementedError("write your pallas kernel here")


# speedup vs baseline: 3.2169x; 3.2169x over previous
import functools

import jax
import jax.numpy as jnp
from jax import lax
from jax.experimental import pallas as pl
from jax.experimental.pallas import tpu as pltpu
from jax.experimental.pallas import tpu_sc as plsc

N = 10000
D = 128
E = 160000

CHUNK = 128
CHUNKS_PER_REL = 1280
E_PAD = CHUNKS_PER_REL * CHUNK
EDGE_PAD = E_PAD - E
STEPS = CHUNKS_PER_REL // 16
ACC_ROWS = 10112
ZROWS = ACC_ROWS // 16
ZERO_ROW = 2 * N
DEG_ROWS = 80


def _sc_segsum_body(table, src, dst, zrow, out_s, out_d,
                    sidx, didx, msgs, degl, idbuf, acc, accd, sem):
    cid = lax.axis_index("c")
    sid = lax.axis_index("s")
    zr = sid * ZROWS
    tail = ZROWS - 4 * CHUNK


    pltpu.sync_copy(zrow, msgs)
    pltpu.sync_copy(zrow.at[pl.ds(0, DEG_ROWS)], degl)

    iota16 = lax.iota(jnp.int32, 16)
    for k in range(DEG_ROWS // 16):
        idbuf[pl.ds(k * 16, 16)] = iota16 + k * 16

    for k in range(4):
        pltpu.sync_copy(msgs, acc.at[pl.ds(zr + k * CHUNK, CHUNK)])
    pltpu.sync_copy(msgs.at[pl.ds(0, tail)], acc.at[pl.ds(zr + 4 * CHUNK, tail)])

    @pl.when(sid == 0)
    def _zero_accd():
        pltpu.sync_copy(msgs.at[pl.ds(0, DEG_ROWS)], accd)

    plsc.subcore_barrier()

    base = (cid * CHUNKS_PER_REL + sid * STEPS) * CHUNK
    ones16 = jnp.ones((16,), jnp.float32)

    def _step(g, carry):
        off = base + g * CHUNK
        pltpu.sync_copy(src.at[pl.ds(off, CHUNK)], sidx)
        pltpu.sync_copy(dst.at[pl.ds(off, CHUNK)], didx)
        pltpu.async_copy(table.at[sidx], msgs, sem).wait()
        pltpu.sync_copy(msgs, acc.at[didx], add=True)
        for j in range(CHUNK // 16):
            idx16 = didx[pl.ds(j * 16, 16)]
            row16 = lax.shift_right_logical(idx16, 7)
            col16 = jnp.bitwise_and(idx16, 127)
            plsc.addupdate_scatter(degl, [row16, col16], ones16)
        return carry
    lax.fori_loop(0, STEPS, _step, 0)

    pltpu.sync_copy(degl, accd.at[idbuf], add=True)

    plsc.subcore_barrier()

    wr = cid * ACC_ROWS + zr
    for k in range(4):
        pltpu.sync_copy(acc.at[pl.ds(zr + k * CHUNK, CHUNK)], msgs)
        pltpu.sync_copy(msgs, out_s.at[pl.ds(wr + k * CHUNK, CHUNK)])
    pltpu.sync_copy(acc.at[pl.ds(zr + 4 * CHUNK, tail)], msgs.at[pl.ds(0, tail)])
    pltpu.sync_copy(msgs.at[pl.ds(0, tail)], out_s.at[pl.ds(wr + 4 * CHUNK, tail)])

    @pl.when(sid == 0)
    def _write_deg():
        pltpu.sync_copy(accd, msgs.at[pl.ds(0, DEG_ROWS)])
        pltpu.sync_copy(msgs.at[pl.ds(0, DEG_ROWS)], out_d.at[cid])


_sc_segsum = functools.partial(
    pl.kernel,
    out_type=[
        jax.ShapeDtypeStruct((2 * ACC_ROWS, D), jnp.float32),
        jax.ShapeDtypeStruct((2, DEG_ROWS, D), jnp.float32),
    ],
    mesh=plsc.VectorSubcoreMesh(core_axis_name="c", subcore_axis_name="s"),
    compiler_params=pltpu.CompilerParams(needs_layout_passes=False),
    scratch_types=[
        pltpu.VMEM((CHUNK,), jnp.int32),
        pltpu.VMEM((CHUNK,), jnp.int32),
        pltpu.VMEM((CHUNK, D), jnp.float32),
        pltpu.VMEM((DEG_ROWS, D), jnp.float32),
        pltpu.VMEM((DEG_ROWS,), jnp.int32),
        pltpu.VMEM_SHARED((ACC_ROWS, D), jnp.float32),
        pltpu.VMEM_SHARED((DEG_ROWS, D), jnp.float32),
        pltpu.SemaphoreType.DMA,
    ],
)(_sc_segsum_body)


def _dense_body(fc_ref, sb_ref, so_ref, degb_ref, dego_ref,
                wec_ref, web_ref, weo_ref, wb1_ref, wo1_ref, wb2_ref, wo2_ref,
                ws1_ref, ws2_ref,
                w1_ref, b1_ref, w2_ref, b2_ref, w3_ref, b3_ref, out_ref):
    f32 = jnp.float32
    sb = sb_ref[...]
    so = so_ref[...]
    db = jnp.maximum(degb_ref[0], 1.0)
    do_ = jnp.maximum(dego_ref[0], 1.0)
    aggb = sb / db
    aggo = so / do_

    web = web_ref[...]
    weo = weo_ref[...]
    cb1 = jnp.dot(web, wb1_ref[...], preferred_element_type=f32)
    co1 = jnp.dot(weo, wo1_ref[...], preferred_element_type=f32)
    cb2 = jnp.dot(web, wb2_ref[...], preferred_element_type=f32)
    co2 = jnp.dot(weo, wo2_ref[...], preferred_element_type=f32)

    hc = jnp.dot(fc_ref[...], wec_ref[...], preferred_element_type=f32)
    mb1 = jnp.dot(aggb, cb1, preferred_element_type=f32)
    mo1 = jnp.dot(aggo, co1, preferred_element_type=f32)
    mb2 = jnp.dot(aggb, cb2, preferred_element_type=f32)
    mo2 = jnp.dot(aggo, co2, preferred_element_type=f32)

    h1 = jnp.maximum(
        jnp.dot(hc, ws1_ref[...], preferred_element_type=f32) + mb1 + mo1, 0.0)
    h2 = jnp.maximum(
        jnp.dot(h1, ws2_ref[...], preferred_element_type=f32) + mb2 + mo2, 0.0)

    w1 = w1_ref[...]
    x = (jnp.dot(h1, w1[0:D], preferred_element_type=f32)
         + jnp.dot(h2, w1[D:2 * D], preferred_element_type=f32)
         + jnp.dot(hc, w1[2 * D:3 * D], preferred_element_type=f32)
         + b1_ref[...])
    x = jnp.dot(x, w2_ref[...], preferred_element_type=f32) + b2_ref[...]
    x = jax.nn.sigmoid(x)
    x = jnp.dot(x, w3_ref[...], preferred_element_type=f32) + b3_ref[...]
    x = jax.nn.sigmoid(x)
    out_ref[...] = x


def _dense(fc, sums, deg, wec, web, weo, wb1, wo1, wb2, wo2, ws1, ws2,
           w1, b1, w2, b2, w3, b3):
    bm = ZROWS
    nblk = ACC_ROWS // bm
    grid = (nblk,)
    full = lambda shape: pl.BlockSpec(shape, lambda m: tuple(0 for _ in shape))
    out = pl.pallas_call(
        _dense_body,
        grid=grid,
        in_specs=[
            pl.BlockSpec((bm, D), lambda m: (m, 0)),
            pl.BlockSpec((bm, D), lambda m: (m, 0)),
            pl.BlockSpec((bm, D), lambda m: (m + nblk, 0)),
            pl.BlockSpec((1, bm, 1), lambda m: (0, m, 0)),
            pl.BlockSpec((1, bm, 1), lambda m: (1, m, 0)),
            full((D, D)), full((D, D)), full((D, D)), full((D, D)),
            full((D, D)), full((D, D)), full((D, D)),
            full((D, D)), full((D, D)),
            full((3 * D, 20)), full((20,)), full((20, 10)), full((10,)),
            full((10, 2)), full((2,)),
        ],
        out_specs=pl.BlockSpec((bm, 2), lambda m: (m, 0)),
        out_shape=jax.ShapeDtypeStruct((ACC_ROWS, 2), jnp.float32),
    )(fc, sums, sums, deg, deg, wec, web, weo, wb1, wo1, wb2, wo2, ws1, ws2,
      w1, b1, w2, b2, w3, b3)
    return out[:N]


def kernel(feat_company, feat_brand, feat_organize, edge_bc, edge_oc,
           W_emb_c, W_emb_b, W_emb_o,
           W_self_1, W_b_1, W_o_1, W_self_2, W_b_2, W_o_2,
           W1, b1, W2, b2, W3, b3):
    i32 = jnp.int32
    f32 = jnp.float32

    table = jnp.concatenate(
        [feat_brand, feat_organize, jnp.zeros((1, D), f32)], axis=0)

    pad_src = jnp.full((EDGE_PAD,), ZERO_ROW, i32)
    pad_dst = jnp.full((EDGE_PAD,), N, i32)
    src_all = jnp.concatenate(
        [edge_bc[0].astype(i32), pad_src,
         edge_oc[0].astype(i32) + N, pad_src])
    dst_all = jnp.concatenate(
        [edge_bc[1].astype(i32), pad_dst,
         edge_oc[1].astype(i32), pad_dst])

    zrow = jnp.zeros((CHUNK, D), f32)

    sums, deg_r = _sc_segsum(table, src_all, dst_all, zrow)
    deg = deg_r.reshape(2, DEG_ROWS * D, 1)[:, :ACC_ROWS]

    fc_pad = jnp.pad(feat_company, ((0, ACC_ROWS - N), (0, 0)))
    return _dense(fc_pad, sums, deg, W_emb_c, W_emb_b, W_emb_o,
                  W_b_1, W_o_1, W_b_2, W_o_2,
                  W_self_1, W_self_2, W1, b1, W2, b2, W3, b3)


# speedup vs baseline: 3.6710x; 1.1412x over previous
import functools

import jax
import jax.numpy as jnp
from jax import lax
from jax.experimental import pallas as pl
from jax.experimental.pallas import tpu as pltpu
from jax.experimental.pallas import tpu_sc as plsc

N = 10000
D = 128
E = 160000

CHUNK = 128
CHUNKS_PER_REL = 1280
E_PAD = CHUNKS_PER_REL * CHUNK
EDGE_PAD = E_PAD - E
STEPS = CHUNKS_PER_REL // 16
ACC_ROWS = 10112
ZROWS = ACC_ROWS // 16
ZERO_ROW = 2 * N
DEG_ROWS = 80


def _sc_segsum_body(table, src, dst, zrow, out_s, out_d,
                    sidx0, didx0, sidx1, didx1, msgs0, msgs1, degl, idbuf,
                    acc, accd, semg0, semg1, sems0, sems1):
    cid = lax.axis_index("c")
    sid = lax.axis_index("s")
    zr = sid * ZROWS
    tail = ZROWS - 4 * CHUNK
    sbuf = (sidx0, sidx1)
    dbuf = (didx0, didx1)
    mbuf = (msgs0, msgs1)
    gsem = (semg0, semg1)
    ssem = (sems0, sems1)


    pltpu.sync_copy(zrow, msgs0)
    pltpu.sync_copy(zrow.at[pl.ds(0, DEG_ROWS)], degl)

    iota16 = lax.iota(jnp.int32, 16)
    for k in range(DEG_ROWS // 16):
        idbuf[pl.ds(k * 16, 16)] = iota16 + k * 16

    for k in range(4):
        pltpu.sync_copy(msgs0, acc.at[pl.ds(zr + k * CHUNK, CHUNK)])
    pltpu.sync_copy(msgs0.at[pl.ds(0, tail)],
                    acc.at[pl.ds(zr + 4 * CHUNK, tail)])

    @pl.when(sid == 0)
    def _zero_accd():
        pltpu.sync_copy(msgs0.at[pl.ds(0, DEG_ROWS)], accd)

    plsc.subcore_barrier()

    base = (cid * CHUNKS_PER_REL + sid * STEPS) * CHUNK
    ones16 = jnp.ones((16,), jnp.float32)

    def _load_idx(g, b):
        off = base + g * CHUNK
        pltpu.sync_copy(src.at[pl.ds(off, CHUNK)], sbuf[b])
        pltpu.sync_copy(dst.at[pl.ds(off, CHUNK)], dbuf[b])

    def _hist(b):
        for j in range(CHUNK // 16):
            idx16 = dbuf[b][pl.ds(j * 16, 16)]
            row16 = lax.shift_right_logical(idx16, 7)
            col16 = jnp.bitwise_and(idx16, 127)
            plsc.addupdate_scatter(degl, [row16, col16], ones16)

    _load_idx(0, 0)
    pltpu.async_copy(table.at[sidx0], msgs0, semg0)

    def _pair(k, carry):
        for b in (0, 1):
            g = 2 * k + b
            q = 1 - b
            if b == 0:
                @pl.when(k > 0)
                def _drain():
                    pltpu.make_async_copy(
                        mbuf[q], acc.at[dbuf[q]], ssem[q]).wait()
                _load_idx(g + 1, q)
                pltpu.async_copy(table.at[sbuf[q]], mbuf[q], gsem[q])
            else:
                pltpu.make_async_copy(mbuf[q], acc.at[dbuf[q]], ssem[q]).wait()

                @pl.when(k < STEPS // 2 - 1)
                def _prefetch():
                    _load_idx(g + 1, q)
                    pltpu.async_copy(table.at[sbuf[q]], mbuf[q], gsem[q])
            pltpu.make_async_copy(table.at[sbuf[b]], mbuf[b], gsem[b]).wait()
            pltpu.async_copy(mbuf[b], acc.at[dbuf[b]], ssem[b], add=True)
            _hist(b)
        return carry
    lax.fori_loop(0, STEPS // 2, _pair, 0)
    pltpu.make_async_copy(mbuf[1], acc.at[dbuf[1]], ssem[1]).wait()

    pltpu.sync_copy(degl, accd.at[idbuf], add=True)

    plsc.subcore_barrier()

    wr = cid * ACC_ROWS + zr
    for k in range(4):
        pltpu.sync_copy(acc.at[pl.ds(zr + k * CHUNK, CHUNK)], msgs0)
        pltpu.sync_copy(msgs0, out_s.at[pl.ds(wr + k * CHUNK, CHUNK)])
    pltpu.sync_copy(acc.at[pl.ds(zr + 4 * CHUNK, tail)],
                    msgs0.at[pl.ds(0, tail)])
    pltpu.sync_copy(msgs0.at[pl.ds(0, tail)],
                    out_s.at[pl.ds(wr + 4 * CHUNK, tail)])

    @pl.when(sid == 0)
    def _write_deg():
        pltpu.sync_copy(accd, msgs0.at[pl.ds(0, DEG_ROWS)])
        pltpu.sync_copy(msgs0.at[pl.ds(0, DEG_ROWS)], out_d.at[cid])


_sc_segsum = functools.partial(
    pl.kernel,
    out_type=[
        jax.ShapeDtypeStruct((2 * ACC_ROWS, D), jnp.float32),
        jax.ShapeDtypeStruct((2, DEG_ROWS, D), jnp.float32),
    ],
    mesh=plsc.VectorSubcoreMesh(core_axis_name="c", subcore_axis_name="s"),
    compiler_params=pltpu.CompilerParams(needs_layout_passes=False),
    scratch_types=[
        pltpu.VMEM((CHUNK,), jnp.int32),
        pltpu.VMEM((CHUNK,), jnp.int32),
        pltpu.VMEM((CHUNK,), jnp.int32),
        pltpu.VMEM((CHUNK,), jnp.int32),
        pltpu.VMEM((CHUNK, D), jnp.float32),
        pltpu.VMEM((CHUNK, D), jnp.float32),
        pltpu.VMEM((DEG_ROWS, D), jnp.float32),
        pltpu.VMEM((DEG_ROWS,), jnp.int32),
        pltpu.VMEM_SHARED((ACC_ROWS, D), jnp.float32),
        pltpu.VMEM_SHARED((DEG_ROWS, D), jnp.float32),
        pltpu.SemaphoreType.DMA,
        pltpu.SemaphoreType.DMA,
        pltpu.SemaphoreType.DMA,
        pltpu.SemaphoreType.DMA,
    ],
)(_sc_segsum_body)


def _dense_body(fc_ref, sb_ref, so_ref, degb_ref, dego_ref,
                wec_ref, web_ref, weo_ref, wb1_ref, wo1_ref, wb2_ref, wo2_ref,
                ws1_ref, ws2_ref,
                w1_ref, b1_ref, w2_ref, b2_ref, w3_ref, b3_ref, out_ref):
    f32 = jnp.float32
    sb = sb_ref[...]
    so = so_ref[...]
    db = jnp.maximum(degb_ref[0], 1.0)
    do_ = jnp.maximum(dego_ref[0], 1.0)
    aggb = sb / db
    aggo = so / do_

    web = web_ref[...]
    weo = weo_ref[...]
    cb1 = jnp.dot(web, wb1_ref[...], preferred_element_type=f32)
    co1 = jnp.dot(weo, wo1_ref[...], preferred_element_type=f32)
    cb2 = jnp.dot(web, wb2_ref[...], preferred_element_type=f32)
    co2 = jnp.dot(weo, wo2_ref[...], preferred_element_type=f32)

    hc = jnp.dot(fc_ref[...], wec_ref[...], preferred_element_type=f32)
    mb1 = jnp.dot(aggb, cb1, preferred_element_type=f32)
    mo1 = jnp.dot(aggo, co1, preferred_element_type=f32)
    mb2 = jnp.dot(aggb, cb2, preferred_element_type=f32)
    mo2 = jnp.dot(aggo, co2, preferred_element_type=f32)

    h1 = jnp.maximum(
        jnp.dot(hc, ws1_ref[...], preferred_element_type=f32) + mb1 + mo1, 0.0)
    h2 = jnp.maximum(
        jnp.dot(h1, ws2_ref[...], preferred_element_type=f32) + mb2 + mo2, 0.0)

    w1 = w1_ref[...]
    x = (jnp.dot(h1, w1[0:D], preferred_element_type=f32)
         + jnp.dot(h2, w1[D:2 * D], preferred_element_type=f32)
         + jnp.dot(hc, w1[2 * D:3 * D], preferred_element_type=f32)
         + b1_ref[...])
    x = jnp.dot(x, w2_ref[...], preferred_element_type=f32) + b2_ref[...]
    x = jax.nn.sigmoid(x)
    x = jnp.dot(x, w3_ref[...], preferred_element_type=f32) + b3_ref[...]
    x = jax.nn.sigmoid(x)
    out_ref[...] = x


def _dense(fc, sums, deg, wec, web, weo, wb1, wo1, wb2, wo2, ws1, ws2,
           w1, b1, w2, b2, w3, b3):
    bm = ZROWS
    nblk = ACC_ROWS // bm
    grid = (nblk,)
    full = lambda shape: pl.BlockSpec(shape, lambda m: tuple(0 for _ in shape))
    out = pl.pallas_call(
        _dense_body,
        grid=grid,
        in_specs=[
            pl.BlockSpec((bm, D), lambda m: (m, 0)),
            pl.BlockSpec((bm, D), lambda m: (m, 0)),
            pl.BlockSpec((bm, D), lambda m: (m + nblk, 0)),
            pl.BlockSpec((1, bm, 1), lambda m: (0, m, 0)),
            pl.BlockSpec((1, bm, 1), lambda m: (1, m, 0)),
            full((D, D)), full((D, D)), full((D, D)), full((D, D)),
            full((D, D)), full((D, D)), full((D, D)),
            full((D, D)), full((D, D)),
            full((3 * D, 20)), full((20,)), full((20, 10)), full((10,)),
            full((10, 2)), full((2,)),
        ],
        out_specs=pl.BlockSpec((bm, 2), lambda m: (m, 0)),
        out_shape=jax.ShapeDtypeStruct((ACC_ROWS, 2), jnp.float32),
    )(fc, sums, sums, deg, deg, wec, web, weo, wb1, wo1, wb2, wo2, ws1, ws2,
      w1, b1, w2, b2, w3, b3)
    return out[:N]


def kernel(feat_company, feat_brand, feat_organize, edge_bc, edge_oc,
           W_emb_c, W_emb_b, W_emb_o,
           W_self_1, W_b_1, W_o_1, W_self_2, W_b_2, W_o_2,
           W1, b1, W2, b2, W3, b3):
    i32 = jnp.int32
    f32 = jnp.float32

    table = jnp.concatenate(
        [feat_brand, feat_organize, jnp.zeros((1, D), f32)], axis=0)

    pad_src = jnp.full((EDGE_PAD,), ZERO_ROW, i32)
    pad_dst = jnp.full((EDGE_PAD,), N, i32)
    src_all = jnp.concatenate(
        [edge_bc[0].astype(i32), pad_src,
         edge_oc[0].astype(i32) + N, pad_src])
    dst_all = jnp.concatenate(
        [edge_bc[1].astype(i32), pad_dst,
         edge_oc[1].astype(i32), pad_dst])

    zrow = jnp.zeros((CHUNK, D), f32)

    sums, deg_r = _sc_segsum(table, src_all, dst_all, zrow)
    deg = deg_r.reshape(2, DEG_ROWS * D, 1)[:, :ACC_ROWS]

    fc_pad = jnp.pad(feat_company, ((0, ACC_ROWS - N), (0, 0)))
    return _dense(fc_pad, sums, deg, W_emb_c, W_emb_b, W_emb_o,
                  W_b_1, W_o_1, W_b_2, W_o_2,
                  W_self_1, W_self_2, W1, b1, W2, b2, W3, b3)


# speedup vs baseline: 3.8640x; 1.0526x over previous
import functools

import jax
import jax.numpy as jnp
from jax import lax
from jax.experimental import pallas as pl
from jax.experimental.pallas import tpu as pltpu
from jax.experimental.pallas import tpu_sc as plsc

N = 10000
D = 128
E = 160000

CHUNK = 128
CHUNKS_PER_REL = 1280
E_PAD = CHUNKS_PER_REL * CHUNK
EDGE_PAD = E_PAD - E
STEPS = CHUNKS_PER_REL // 16
ACC_ROWS = 10112
ZROWS = ACC_ROWS // 16
ZERO_ROW = 2 * N
DEG_ROWS = 80


def _sc_segsum_body(table, src, dst, zrow, out_s, out_d,
                    sidx0, didx0, sidx1, didx1, didx2, didx3, msgs0, msgs1,
                    degl, idbuf, acc, accd,
                    semg0, semg1, sems0, sems1, isem0, isem1):
    cid = lax.axis_index("c")
    sid = lax.axis_index("s")
    zr = sid * ZROWS
    tail = ZROWS - 4 * CHUNK
    sbuf = (sidx0, sidx1)
    dbuf = (didx0, didx1, didx2, didx3)
    mbuf = (msgs0, msgs1)
    gsem = (semg0, semg1)
    ssem = (sems0, sems1)
    isem = (isem0, isem1)


    pltpu.sync_copy(zrow, msgs0)
    pltpu.sync_copy(zrow.at[pl.ds(0, DEG_ROWS)], degl)

    iota16 = lax.iota(jnp.int32, 16)
    for k in range(DEG_ROWS // 16):
        idbuf[pl.ds(k * 16, 16)] = iota16 + k * 16

    for k in range(4):
        pltpu.sync_copy(msgs0, acc.at[pl.ds(zr + k * CHUNK, CHUNK)])
    pltpu.sync_copy(msgs0.at[pl.ds(0, tail)],
                    acc.at[pl.ds(zr + 4 * CHUNK, tail)])

    @pl.when(sid == 0)
    def _zero_accd():
        pltpu.sync_copy(msgs0.at[pl.ds(0, DEG_ROWS)], accd)

    plsc.subcore_barrier()

    base = (cid * CHUNKS_PER_REL + sid * STEPS) * CHUNK
    ones16 = jnp.ones((16,), jnp.float32)

    def _load_idx(g, b):
        off = base + g * CHUNK
        pltpu.sync_copy(src.at[pl.ds(off, CHUNK)], sbuf[b])
        pltpu.sync_copy(dst.at[pl.ds(off, CHUNK)], dbuf[b])

    def _hist(b):
        for j in range(CHUNK // 16):
            idx16 = dbuf[b][pl.ds(j * 16, 16)]
            row16 = lax.shift_right_logical(idx16, 7)
            col16 = jnp.bitwise_and(idx16, 127)
            plsc.addupdate_scatter(degl, [row16, col16], ones16)

    def _idx_start(g, sp, dr):
        off = base + g * CHUNK
        sem = isem[sp]
        pltpu.async_copy(src.at[pl.ds(off, CHUNK)], sbuf[sp], sem)
        pltpu.async_copy(dst.at[pl.ds(off, CHUNK)], dbuf[dr], sem)

    def _idx_wait(g, sp, dr):
        off = base + g * CHUNK
        sem = isem[sp]
        pltpu.make_async_copy(src.at[pl.ds(off, CHUNK)], sbuf[sp], sem).wait()
        pltpu.make_async_copy(dst.at[pl.ds(off, CHUNK)], dbuf[dr], sem).wait()

    _load_idx(0, 0)
    _idx_start(1, 1, 1)
    pltpu.async_copy(table.at[sidx0], msgs0, semg0)

    NK = STEPS // 4

    def _quad(k, carry):
        k4 = 4 * k
        for b in range(4):
            p = b % 2
            q = 1 - p

            def _drain_q():
                pltpu.make_async_copy(
                    mbuf[q], acc.at[dbuf[(b + 3) % 4]], ssem[q]).wait()

            def _launch_next():
                _idx_wait(k4 + b + 1, q, (b + 1) % 4)
                pltpu.async_copy(table.at[sbuf[q]], mbuf[q], gsem[q])

            def _prefetch_idx():
                _idx_start(k4 + b + 2, p, (b + 2) % 4)

            if b == 0:
                @pl.when(k > 0)
                def _():
                    _drain_q()
                _launch_next()
            elif b == 3:
                _drain_q()

                @pl.when(k < NK - 1)
                def _():
                    _launch_next()
            else:
                _drain_q()
                _launch_next()

            pltpu.make_async_copy(table.at[sbuf[p]], mbuf[p], gsem[p]).wait()
            pltpu.async_copy(mbuf[p], acc.at[dbuf[b]], ssem[p], add=True)
            if b >= 2:
                @pl.when(k < NK - 1)
                def _():
                    _prefetch_idx()
            else:
                _prefetch_idx()
            _hist(b)
        return carry
    lax.fori_loop(0, NK, _quad, 0)
    pltpu.make_async_copy(mbuf[1], acc.at[dbuf[3]], ssem[1]).wait()

    pltpu.sync_copy(degl, accd.at[idbuf], add=True)

    plsc.subcore_barrier()

    wr = cid * ACC_ROWS + zr
    for k in range(4):
        pltpu.sync_copy(acc.at[pl.ds(zr + k * CHUNK, CHUNK)], msgs0)
        pltpu.sync_copy(msgs0, out_s.at[pl.ds(wr + k * CHUNK, CHUNK)])
    pltpu.sync_copy(acc.at[pl.ds(zr + 4 * CHUNK, tail)],
                    msgs0.at[pl.ds(0, tail)])
    pltpu.sync_copy(msgs0.at[pl.ds(0, tail)],
                    out_s.at[pl.ds(wr + 4 * CHUNK, tail)])

    @pl.when(sid == 0)
    def _write_deg():
        pltpu.sync_copy(accd, msgs0.at[pl.ds(0, DEG_ROWS)])
        pltpu.sync_copy(msgs0.at[pl.ds(0, DEG_ROWS)], out_d.at[cid])


_sc_segsum = functools.partial(
    pl.kernel,
    out_type=[
        jax.ShapeDtypeStruct((2 * ACC_ROWS, D), jnp.float32),
        jax.ShapeDtypeStruct((2, DEG_ROWS, D), jnp.float32),
    ],
    mesh=plsc.VectorSubcoreMesh(core_axis_name="c", subcore_axis_name="s"),
    compiler_params=pltpu.CompilerParams(needs_layout_passes=False),
    scratch_types=[
        pltpu.VMEM((CHUNK,), jnp.int32),
        pltpu.VMEM((CHUNK,), jnp.int32),
        pltpu.VMEM((CHUNK,), jnp.int32),
        pltpu.VMEM((CHUNK,), jnp.int32),
        pltpu.VMEM((CHUNK,), jnp.int32),
        pltpu.VMEM((CHUNK,), jnp.int32),
        pltpu.VMEM((CHUNK, D), jnp.float32),
        pltpu.VMEM((CHUNK, D), jnp.float32),
        pltpu.VMEM((DEG_ROWS, D), jnp.float32),
        pltpu.VMEM((DEG_ROWS,), jnp.int32),
        pltpu.VMEM_SHARED((ACC_ROWS, D), jnp.float32),
        pltpu.VMEM_SHARED((DEG_ROWS, D), jnp.float32),
        pltpu.SemaphoreType.DMA,
        pltpu.SemaphoreType.DMA,
        pltpu.SemaphoreType.DMA,
        pltpu.SemaphoreType.DMA,
        pltpu.SemaphoreType.DMA,
        pltpu.SemaphoreType.DMA,
    ],
)(_sc_segsum_body)


def _dense_body(fc_ref, sb_ref, so_ref, degb_ref, dego_ref,
                wec_ref, web_ref, weo_ref, wb1_ref, wo1_ref, wb2_ref, wo2_ref,
                ws1_ref, ws2_ref,
                w1_ref, b1_ref, w2_ref, b2_ref, w3_ref, b3_ref, out_ref):
    f32 = jnp.float32
    sb = sb_ref[...]
    so = so_ref[...]
    db = jnp.maximum(degb_ref[0], 1.0)
    do_ = jnp.maximum(dego_ref[0], 1.0)
    aggb = sb / db
    aggo = so / do_

    web = web_ref[...]
    weo = weo_ref[...]
    cb1 = jnp.dot(web, wb1_ref[...], preferred_element_type=f32)
    co1 = jnp.dot(weo, wo1_ref[...], preferred_element_type=f32)
    cb2 = jnp.dot(web, wb2_ref[...], preferred_element_type=f32)
    co2 = jnp.dot(weo, wo2_ref[...], preferred_element_type=f32)

    hc = jnp.dot(fc_ref[...], wec_ref[...], preferred_element_type=f32)
    mb1 = jnp.dot(aggb, cb1, preferred_element_type=f32)
    mo1 = jnp.dot(aggo, co1, preferred_element_type=f32)
    mb2 = jnp.dot(aggb, cb2, preferred_element_type=f32)
    mo2 = jnp.dot(aggo, co2, preferred_element_type=f32)

    h1 = jnp.maximum(
        jnp.dot(hc, ws1_ref[...], preferred_element_type=f32) + mb1 + mo1, 0.0)
    h2 = jnp.maximum(
        jnp.dot(h1, ws2_ref[...], preferred_element_type=f32) + mb2 + mo2, 0.0)

    w1 = w1_ref[...]
    x = (jnp.dot(h1, w1[0:D], preferred_element_type=f32)
         + jnp.dot(h2, w1[D:2 * D], preferred_element_type=f32)
         + jnp.dot(hc, w1[2 * D:3 * D], preferred_element_type=f32)
         + b1_ref[...])
    x = jnp.dot(x, w2_ref[...], preferred_element_type=f32) + b2_ref[...]
    x = jax.nn.sigmoid(x)
    x = jnp.dot(x, w3_ref[...], preferred_element_type=f32) + b3_ref[...]
    x = jax.nn.sigmoid(x)
    out_ref[...] = x


def _dense(fc, sums, deg, wec, web, weo, wb1, wo1, wb2, wo2, ws1, ws2,
           w1, b1, w2, b2, w3, b3):
    bm = ZROWS
    nblk = ACC_ROWS // bm
    grid = (nblk,)
    full = lambda shape: pl.BlockSpec(shape, lambda m: tuple(0 for _ in shape))
    out = pl.pallas_call(
        _dense_body,
        grid=grid,
        in_specs=[
            pl.BlockSpec((bm, D), lambda m: (m, 0)),
            pl.BlockSpec((bm, D), lambda m: (m, 0)),
            pl.BlockSpec((bm, D), lambda m: (m + nblk, 0)),
            pl.BlockSpec((1, bm, 1), lambda m: (0, m, 0)),
            pl.BlockSpec((1, bm, 1), lambda m: (1, m, 0)),
            full((D, D)), full((D, D)), full((D, D)), full((D, D)),
            full((D, D)), full((D, D)), full((D, D)),
            full((D, D)), full((D, D)),
            full((3 * D, 20)), full((20,)), full((20, 10)), full((10,)),
            full((10, 2)), full((2,)),
        ],
        out_specs=pl.BlockSpec((bm, 2), lambda m: (m, 0)),
        out_shape=jax.ShapeDtypeStruct((ACC_ROWS, 2), jnp.float32),
    )(fc, sums, sums, deg, deg, wec, web, weo, wb1, wo1, wb2, wo2, ws1, ws2,
      w1, b1, w2, b2, w3, b3)
    return out[:N]


def kernel(feat_company, feat_brand, feat_organize, edge_bc, edge_oc,
           W_emb_c, W_emb_b, W_emb_o,
           W_self_1, W_b_1, W_o_1, W_self_2, W_b_2, W_o_2,
           W1, b1, W2, b2, W3, b3):
    i32 = jnp.int32
    f32 = jnp.float32

    table = jnp.concatenate(
        [feat_brand, feat_organize, jnp.zeros((1, D), f32)], axis=0)

    pad_src = jnp.full((EDGE_PAD,), ZERO_ROW, i32)
    pad_dst = jnp.full((EDGE_PAD,), N, i32)
    src_all = jnp.concatenate(
        [edge_bc[0].astype(i32), pad_src,
         edge_oc[0].astype(i32) + N, pad_src])
    dst_all = jnp.concatenate(
        [edge_bc[1].astype(i32), pad_dst,
         edge_oc[1].astype(i32), pad_dst])

    zrow = jnp.zeros((CHUNK, D), f32)

    sums, deg_r = _sc_segsum(table, src_all, dst_all, zrow)
    deg = deg_r.reshape(2, DEG_ROWS * D, 1)[:, :ACC_ROWS]

    fc_pad = jnp.pad(feat_company, ((0, ACC_ROWS - N), (0, 0)))
    return _dense(fc_pad, sums, deg, W_emb_c, W_emb_b, W_emb_o,
                  W_b_1, W_o_1, W_b_2, W_o_2,
                  W_self_1, W_self_2, W1, b1, W2, b2, W3, b3)
